# Initial kernel scaffold; baseline (speedup 1.0000x reference)
import functools

import jax
import jax.numpy as jnp
from jax import lax
from jax.experimental import pallas as pl
from jax.experimental.pallas import tpu as pltpu
from jax.experimental.pallas import tpu_sc as plsc

NSC = 2
NSUB = 16
LANES = 16
NW = NSC * NSUB

BN_EPS = 1e-5
F32 = jnp.float32


@functools.cache
def _make_sc_aggr(N, E, H, ED, C):
    EPW = E // NW
    NCHUNK = EPW // C
    RPT = N // NSUB
    ZR = 125 if RPT % 125 == 0 else RPT
    NZ = RPT // ZR
    NPAD = N + 16
    HL = H // LANES

    mesh = plsc.VectorSubcoreMesh(core_axis_name="c", subcore_axis_name="s")

    @functools.partial(
        pl.kernel,
        out_type=jax.ShapeDtypeStruct((NSC, N, H), F32),
        mesh=mesh,
        scratch_types=[
            pltpu.VMEM((ED, H), F32),
            pltpu.VMEM((H,), F32),
            pltpu.VMEM((C,), jnp.int32),
            pltpu.VMEM((C,), jnp.int32),
            pltpu.VMEM((C, ED), F32),
            pltpu.VMEM((C, H), F32),
            pltpu.VMEM((ZR, H), F32),
            pltpu.VMEM_SHARED((NPAD, H), F32),
            pltpu.SemaphoreType.DMA,
        ],
    )
    def sc_aggr(h_hbm, src_hbm, dst_hbm, ea_hbm, ew_hbm, eb_hbm, out_hbm,
                ew_v, eb_v, src_v, dst_v, ea_v, rows_v, zb_v, accum, sem):
        cid = lax.axis_index("c")
        sid = lax.axis_index("s")
        wid = sid * NSC + cid

        pltpu.sync_copy(ew_hbm, ew_v)
        pltpu.sync_copy(eb_hbm, eb_v)

        zvec = jnp.zeros((LANES,), F32)

        def zrow(i, carry):
            for j in range(HL):
                zb_v[i, pl.ds(j * LANES, LANES)] = zvec
            return carry

        lax.fori_loop(0, ZR, zrow, 0)
        r0 = sid * RPT
        for q in range(NZ):
            pltpu.sync_copy(zb_v, accum.at[pl.ds(r0 + q * ZR, ZR), :])
        plsc.subcore_barrier()

        ebs = [eb_v[pl.ds(j * LANES, LANES)] for j in range(HL)]
        ews = [[ew_v[m, pl.ds(j * LANES, LANES)] for j in range(HL)]
               for m in range(ED)]

        def chunk(k, carry):
            off = pl.multiple_of(wid * EPW + k * C, 8)
            pltpu.sync_copy(src_hbm.at[pl.ds(off, C)], src_v)
            pltpu.sync_copy(dst_hbm.at[pl.ds(off, C)], dst_v)
            pltpu.sync_copy(ea_hbm.at[pl.ds(off, C), :], ea_v)
            pltpu.async_copy(h_hbm.at[src_v], rows_v, sem).wait()

            def edge(i, c2):
                a = [ea_v[i, m] for m in range(ED)]
                for j in range(HL):
                    sl = pl.ds(j * LANES, LANES)
                    e = ebs[j]
                    for m in range(ED):
                        e = e + a[m] * ews[m][j]
                    rows_v[i, sl] = jnp.maximum(rows_v[i, sl] + e, 0.0)
                return c2

            lax.fori_loop(0, C, edge, 0)
            pltpu.sync_copy(rows_v, accum.at[dst_v], add=True)
            return carry

        lax.fori_loop(0, NCHUNK, chunk, 0)
        plsc.subcore_barrier()

        for q in range(NZ):
            base = r0 + q * ZR
            pltpu.sync_copy(accum.at[pl.ds(base, ZR), :], zb_v)
            pltpu.sync_copy(zb_v, out_hbm.at[cid, pl.ds(base, ZR), :])

    return sc_aggr


def _encoder_body(x_ref, w_ref, b_ref, o_ref):
    o_ref[...] = (
        jnp.dot(x_ref[...], w_ref[...], preferred_element_type=F32) + b_ref[...]
    )


def _layer_body(h_ref, a0_ref, a1_ref, w1_ref, b1_ref, w2_ref, b2_ref,
                g_ref, be_ref, o_ref):
    z = h_ref[...] + a0_ref[...] + a1_ref[...]
    t = jnp.dot(z, w1_ref[...], preferred_element_type=F32) + b1_ref[...]
    t = jnp.maximum(t, 0.0)
    z2 = jnp.dot(t, w2_ref[...], preferred_element_type=F32) + b2_ref[...]
    n = z2.shape[0]
    mu = jnp.sum(z2, axis=0, keepdims=True) * (1.0 / n)
    d = z2 - mu
    var = jnp.sum(d * d, axis=0, keepdims=True) * (1.0 / n)
    zn = d * lax.rsqrt(var + BN_EPS) * g_ref[...] + be_ref[...]
    o_ref[...] = jnp.maximum(zn, 0.0)


def _readout_body(h_ref, m_ref, w1_ref, b1_ref, pa_ref, w2_ref, b2_ref, o_ref):
    sel = h_ref[...] * m_ref[...]
    y = jnp.dot(sel, w1_ref[...], preferred_element_type=F32) + b1_ref[...]
    y = jnp.where(y >= 0.0, y, pa_ref[0, 0] * y)
    o_ref[...] = (
        jnp.dot(y, w2_ref[...], preferred_element_type=F32) + b2_ref[...]
    )


def _tc_call(body, out_shape, *args):
    return pl.pallas_call(body, out_shape=out_shape)(*args)


def kernel(x, edge_attr, edge_index, mask, nW, nb, eW, eb, W1, b1, W2, b2,
           g, beta, rW1, rb1, pa, rW2, rb2):
    N, D = x.shape
    E, ED = edge_attr.shape
    H = nW.shape[1]
    L = eW.shape[0]

    C = 80
    q = NW * C
    Epad = (-E) % q
    src = edge_index[0]
    dst = edge_index[1]
    ea = edge_attr
    if Epad:
        src = jnp.concatenate([src, jnp.zeros((Epad,), jnp.int32)])
        dst = jnp.concatenate([dst, jnp.full((Epad,), N, jnp.int32)])
        ea = jnp.concatenate([ea, jnp.zeros((Epad, ED), F32)])

    sc_aggr = _make_sc_aggr(N, E + Epad, H, ED, C)

    h = _tc_call(_encoder_body, jax.ShapeDtypeStruct((N, H), F32),
                 x, nW, nb.reshape(1, H))

    for l in range(L):
        parts = sc_aggr(h, src, dst, ea, eW[l], eb[l])
        h = _tc_call(
            _layer_body, jax.ShapeDtypeStruct((N, H), F32),
            h, parts[0], parts[1], W1[l], b1[l].reshape(1, -1),
            W2[l], b2[l].reshape(1, H), g[l].reshape(1, H),
            beta[l].reshape(1, H))

    maskf = mask.astype(F32).reshape(N, 1)
    out = _tc_call(
        _readout_body, jax.ShapeDtypeStruct((N, 1), F32),
        h, maskf, rW1, rb1.reshape(1, -1), pa.reshape(1, 1),
        rW2, rb2.reshape(1, 1))
    return out[:, 0]


# speedup vs baseline: 2.2946x; 2.2946x over previous
import functools

import jax
import jax.numpy as jnp
from jax import lax
from jax.experimental import pallas as pl
from jax.experimental.pallas import tpu as pltpu
from jax.experimental.pallas import tpu_sc as plsc

NSC = 2
NSUB = 16
LANES = 16
NW = NSC * NSUB

BN_EPS = 1e-5
F32 = jnp.float32


def _bf16_round(x):
    return x.astype(jnp.bfloat16).astype(F32)


@functools.cache
def _make_sc_aggr(N, E, H, ED, C):
    EPW = E // NW
    NCHUNK = EPW // C
    ZR = 80 if N % 80 == 0 else 8
    NB = N // ZR
    NQ = (NB + NSUB - 1) // NSUB
    NPAD = N + 16
    HL = H // LANES

    mesh = plsc.VectorSubcoreMesh(core_axis_name="c", subcore_axis_name="s")

    @functools.partial(
        pl.kernel,
        out_type=jax.ShapeDtypeStruct((NSC, N, H), F32),
        mesh=mesh,
        scratch_types=[
            pltpu.VMEM((ED, H), F32),
            pltpu.VMEM((H,), F32),
            pltpu.VMEM((C,), jnp.int32),
            pltpu.VMEM((C,), jnp.int32),
            pltpu.VMEM((C * ED + LANES, ), F32),
            pltpu.VMEM((C, H), F32),
            pltpu.VMEM((ZR, H), F32),
            pltpu.VMEM_SHARED((NPAD, H), F32),
            pltpu.SemaphoreType.DMA,
        ],
    )
    def sc_aggr(h_hbm, src_hbm, dst_hbm, ea_hbm, ew_hbm, eb_hbm, out_hbm,
                ew_v, eb_v, src_v, dst_v, ea_v, rows_v, zb_v, accum, sem):
        cid = lax.axis_index("c")
        sid = lax.axis_index("s")
        wid = sid * NSC + cid

        pltpu.sync_copy(ew_hbm, ew_v)
        pltpu.sync_copy(eb_hbm, eb_v)

        zvec = jnp.zeros((LANES,), F32)

        def zrow(i, carry):
            for j in range(HL):
                zb_v[i, pl.ds(j * LANES, LANES)] = zvec
            return carry

        lax.fori_loop(0, ZR, zrow, 0)
        for q in range(NQ):
            b = q * NSUB + sid

            @pl.when(b < NB)
            def _():
                pltpu.sync_copy(
                    zb_v, accum.at[pl.ds(pl.multiple_of(b * ZR, 8), ZR), :])
        plsc.subcore_barrier()

        ebs = [eb_v[pl.ds(j * LANES, LANES)] for j in range(HL)]
        ews = [[ew_v[m, pl.ds(j * LANES, LANES)] for j in range(HL)]
               for m in range(ED)]

        def chunk(k, carry):
            off = pl.multiple_of(wid * EPW + k * C, 8)
            pltpu.sync_copy(src_hbm.at[pl.ds(off, C)], src_v)
            pltpu.sync_copy(dst_hbm.at[pl.ds(off, C)], dst_v)
            pltpu.sync_copy(ea_hbm.at[pl.ds(off * ED, C * ED)],
                            ea_v.at[pl.ds(0, C * ED)])
            pltpu.async_copy(h_hbm.at[src_v], rows_v, sem).wait()

            def edge(i, c2):
                av = ea_v[pl.ds(i * ED, LANES)]
                a = [av[m] for m in range(ED)]
                for j in range(HL):
                    sl = pl.ds(j * LANES, LANES)
                    e = a[0] * ews[0][j]
                    for m in range(1, ED):
                        e = e + a[m] * ews[m][j]
                    e = e + ebs[j]
                    rows_v[i, sl] = jnp.maximum(rows_v[i, sl] + e, 0.0)
                return c2

            lax.fori_loop(0, C, edge, 0)
            pltpu.sync_copy(rows_v, accum.at[dst_v], add=True)
            return carry

        lax.fori_loop(0, NCHUNK, chunk, 0)
        plsc.subcore_barrier()

        for q in range(NQ):
            b = q * NSUB + sid

            @pl.when(b < NB)
            def _():
                base = pl.multiple_of(b * ZR, 8)
                pltpu.sync_copy(accum.at[pl.ds(base, ZR), :],
                                out_hbm.at[cid, pl.ds(base, ZR), :])

    return sc_aggr


def _encoder_body(x_ref, w_ref, b_ref, o_ref):
    o_ref[...] = (
        jnp.dot(x_ref[...], w_ref[...], preferred_element_type=F32) + b_ref[...]
    )


def _layer_body(h_ref, a0_ref, a1_ref, w1_ref, b1_ref, w2_ref, b2_ref,
                g_ref, be_ref, o_ref):
    z = h_ref[...] + a0_ref[...] + a1_ref[...]
    t = jnp.dot(z, w1_ref[...], preferred_element_type=F32) + b1_ref[...]
    t = jnp.maximum(t, 0.0)
    z2 = jnp.dot(t, w2_ref[...], preferred_element_type=F32) + b2_ref[...]
    n = z2.shape[0]
    mu = jnp.sum(z2, axis=0, keepdims=True) * (1.0 / n)
    d = z2 - mu
    var = jnp.sum(d * d, axis=0, keepdims=True) * (1.0 / n)
    zn = d * lax.rsqrt(var + BN_EPS) * g_ref[...] + be_ref[...]
    o_ref[...] = jnp.maximum(zn, 0.0)


def _readout_body(h_ref, m_ref, w1_ref, b1_ref, pa_ref, w2_ref, b2_ref, o_ref):
    sel = h_ref[...] * m_ref[...]
    y = jnp.dot(sel, w1_ref[...], preferred_element_type=F32) + b1_ref[...]
    y = jnp.where(y >= 0.0, y, pa_ref[0, 0] * y)
    o_ref[...] = (
        jnp.dot(y, w2_ref[...], preferred_element_type=F32) + b2_ref[...]
    )


def _tc_call(body, out_shape, *args):
    return pl.pallas_call(body, out_shape=out_shape)(*args)


def kernel(x, edge_attr, edge_index, mask, nW, nb, eW, eb, W1, b1, W2, b2,
           g, beta, rW1, rb1, pa, rW2, rb2):
    N, D = x.shape
    E, ED = edge_attr.shape
    H = nW.shape[1]
    L = eW.shape[0]

    C = 80
    q = NW * C
    Epad = (-E) % q
    src = edge_index[0]
    dst = edge_index[1]
    ea = edge_attr
    if Epad:
        src = jnp.concatenate([src, jnp.zeros((Epad,), jnp.int32)])
        dst = jnp.concatenate([dst, jnp.full((Epad,), N, jnp.int32)])
        ea = jnp.concatenate([ea, jnp.zeros((Epad, ED), F32)])

    ea = _bf16_round(ea.reshape(-1))
    eW_r = _bf16_round(eW)
    sc_aggr = _make_sc_aggr(N, E + Epad, H, ED, C)

    h = _tc_call(_encoder_body, jax.ShapeDtypeStruct((N, H), F32),
                 x, nW, nb.reshape(1, H))

    for l in range(L):
        parts = sc_aggr(h, src, dst, ea, eW_r[l], eb[l])
        h = _tc_call(
            _layer_body, jax.ShapeDtypeStruct((N, H), F32),
            h, parts[0], parts[1], W1[l], b1[l].reshape(1, -1),
            W2[l], b2[l].reshape(1, H), g[l].reshape(1, H),
            beta[l].reshape(1, H))

    maskf = mask.astype(F32).reshape(N, 1)
    out = _tc_call(
        _readout_body, jax.ShapeDtypeStruct((N, 1), F32),
        h, maskf, rW1, rb1.reshape(1, -1), pa.reshape(1, 1),
        rW2, rb2.reshape(1, 1))
    return out[:, 0]
